# baseline (speedup 1.0000x reference)
import functools

import jax
import jax.numpy as jnp
from jax import lax
from jax.experimental import pallas as pl
from jax.experimental.pallas import tpu as pltpu
from jax.experimental.pallas import tpu_sc as plsc

EMBED_DIM = 64
NUM_WORKERS = 32
NBUF = 4


@functools.lru_cache(maxsize=None)
def _make_sc_kernel(batch: int, seq: int):
    bat_per_w = batch // NUM_WORKERS
    n_groups = bat_per_w // NBUF
    half = seq // 2
    mesh = plsc.VectorSubcoreMesh(core_axis_name="c", subcore_axis_name="s")

    @functools.partial(
        pl.kernel,
        mesh=mesh,
        compiler_params=pltpu.CompilerParams(use_tc_tiling_on_sc=False),
        out_type=jax.ShapeDtypeStruct((batch, half, 2 * EMBED_DIM), jnp.float32),
        scratch_types=[
            pltpu.VMEM((bat_per_w, 2, half), jnp.int32),
            pltpu.VMEM((2, half, EMBED_DIM), jnp.float32),
        ]
        + [pltpu.VMEM((2, half, EMBED_DIM), jnp.float32) for _ in range(NBUF)]
        + [pltpu.SemaphoreType.DMA for _ in range(2 * NBUF)],
    )
    def k(ids_hbm, tok_hbm, pos_hbm, out_hbm, idx_v, pos_v, *rest):
        bufs = rest[:NBUF]
        sg = rest[NBUF : 2 * NBUF]
        so = rest[2 * NBUF : 3 * NBUF]
        wid = lax.axis_index("s") * 2 + lax.axis_index("c")
        bat0 = wid * bat_per_w
        pltpu.sync_copy(pos_hbm, pos_v)
        pltpu.sync_copy(ids_hbm.at[pl.ds(bat0, bat_per_w)], idx_v)

        def gather_start(b, g):
            pltpu.async_copy(tok_hbm.at[idx_v.at[g, 0]], bufs[b].at[0], sg[b])
            pltpu.async_copy(tok_hbm.at[idx_v.at[g, 1]], bufs[b].at[1], sg[b])

        def gather_wait(b, g):
            pltpu.make_async_copy(
                tok_hbm.at[idx_v.at[g, 0]], bufs[b].at[0], sg[b]
            ).wait()
            pltpu.make_async_copy(
                tok_hbm.at[idx_v.at[g, 1]], bufs[b].at[1], sg[b]
            ).wait()

        def out_start(b, g):
            pltpu.async_copy(
                bufs[b].at[0], out_hbm.at[bat0 + g, :, pl.ds(0, EMBED_DIM)], so[b]
            )
            pltpu.async_copy(
                bufs[b].at[1],
                out_hbm.at[bat0 + g, :, pl.ds(EMBED_DIM, EMBED_DIM)],
                so[b],
            )

        def out_wait(b, g):
            pltpu.make_async_copy(
                bufs[b].at[0], out_hbm.at[bat0 + g, :, pl.ds(0, EMBED_DIM)], so[b]
            ).wait()
            pltpu.make_async_copy(
                bufs[b].at[1],
                out_hbm.at[bat0 + g, :, pl.ds(EMBED_DIM, EMBED_DIM)],
                so[b],
            ).wait()

        for b in range(NBUF):
            gather_start(b, b)

        def group_body(i, carry):
            go = i * NBUF
            for b in range(NBUF):
                g = go + b
                gather_wait(b, g)

                def add_body(r, c2, b=b):
                    for eo in range(2):
                        for c in range(EMBED_DIM // 16):
                            sl = pl.ds(c * 16, 16)
                            plsc.addupdate(
                                bufs[b].at[eo, r, sl], pos_v[eo, r, sl]
                            )
                    return c2

                lax.fori_loop(0, half, add_body, None)
                out_start(b, g)

            @pl.when(i < n_groups - 1)
            def _():
                for b in range(NBUF):
                    g = go + b
                    out_wait(b, g)
                    gather_start(b, g + NBUF)

            return carry

        lax.fori_loop(0, n_groups, group_body, None)
        for b in range(NBUF):
            out_wait(b, bat_per_w - NBUF + b)

    return k


from jax.experimental.layout import Format, Layout


@functools.partial(
    jax.jit,
    out_shardings=Format(
        Layout(major_to_minor=(0, 1, 2)),
        jax.sharding.SingleDeviceSharding(jax.devices()[0]),
    ),
)
def kernel(input_ids, token_table, position_table):
    batch, seq = input_ids.shape
    vocab = token_table.shape[0]
    half = seq // 2
    ids2 = input_ids.astype(jnp.int32) * 2
    ids_eo = jnp.stack([ids2[:, 0::2], ids2[:, 1::2]], axis=1)
    tok2 = jnp.pad(token_table, ((0, 0), (0, EMBED_DIM))).reshape(
        2 * vocab, EMBED_DIM
    )
    posf = position_table[:seq].astype(jnp.float32)
    pos = jnp.stack([posf[0::2], posf[1::2]], axis=0)
    out = _make_sc_kernel(batch, seq)(ids_eo, tok2, pos)
    return out.reshape(batch, seq, EMBED_DIM)


# speedup vs baseline: 1.0011x; 1.0011x over previous
import functools

import jax
import jax.numpy as jnp
from jax import lax
from jax.experimental import pallas as pl
from jax.experimental.pallas import tpu as pltpu
from jax.experimental.pallas import tpu_sc as plsc

EMBED_DIM = 64
NUM_WORKERS = 32
NBUF = 4


@functools.lru_cache(maxsize=None)
def _make_sc_kernel(batch: int, seq: int):
    bat_per_w = batch // NUM_WORKERS
    n_groups = bat_per_w // NBUF
    half = seq // 2
    mesh = plsc.VectorSubcoreMesh(core_axis_name="c", subcore_axis_name="s")

    @functools.partial(
        pl.kernel,
        mesh=mesh,
        compiler_params=pltpu.CompilerParams(use_tc_tiling_on_sc=False),
        out_type=jax.ShapeDtypeStruct((batch, half, 2 * EMBED_DIM), jnp.float32),
        scratch_types=[
            pltpu.VMEM((bat_per_w, 2, half), jnp.int32),
            pltpu.VMEM((2, half, EMBED_DIM), jnp.float32),
        ]
        + [pltpu.VMEM((2, half, EMBED_DIM), jnp.float32) for _ in range(NBUF)]

        + [pltpu.SemaphoreType.DMA for _ in range(2 * NBUF)],
    )
    def k(ids_hbm, tok_hbm, pos_hbm, out_hbm, idx_v, pos_v, *rest):
        bufs = rest[:NBUF]
        sg = rest[NBUF : 2 * NBUF]
        so = rest[2 * NBUF : 3 * NBUF]
        wid = lax.axis_index("s") * 2 + lax.axis_index("c")
        bat0 = wid * bat_per_w
        pltpu.sync_copy(pos_hbm, pos_v)
        pltpu.sync_copy(ids_hbm.at[pl.ds(bat0, bat_per_w)], idx_v)


        def gather_start(b, g):
            pltpu.async_copy(tok_hbm.at[idx_v.at[g, 0]], bufs[b].at[0], sg[b])
            pltpu.async_copy(tok_hbm.at[idx_v.at[g, 1]], bufs[b].at[1], sg[b])

        def gather_wait(b, g):
            pltpu.make_async_copy(
                tok_hbm.at[idx_v.at[g, 0]], bufs[b].at[0], sg[b]
            ).wait()
            pltpu.make_async_copy(
                tok_hbm.at[idx_v.at[g, 1]], bufs[b].at[1], sg[b]
            ).wait()

        def out_start(b, g):
            pltpu.async_copy(
                bufs[b].at[0], out_hbm.at[bat0 + g, :, pl.ds(0, EMBED_DIM)], so[b]
            )
            pltpu.async_copy(
                bufs[b].at[1],
                out_hbm.at[bat0 + g, :, pl.ds(EMBED_DIM, EMBED_DIM)],
                so[b],
            )

        def out_wait(b, g):
            pltpu.make_async_copy(
                bufs[b].at[0], out_hbm.at[bat0 + g, :, pl.ds(0, EMBED_DIM)], so[b]
            ).wait()
            pltpu.make_async_copy(
                bufs[b].at[1],
                out_hbm.at[bat0 + g, :, pl.ds(EMBED_DIM, EMBED_DIM)],
                so[b],
            ).wait()

        for b in range(NBUF):
            gather_start(b, b)

        def group_body(i, carry):
            go = i * NBUF
            for b in range(NBUF):
                g = go + b
                gather_wait(b, g)

                def add_body(r, c2, b=b):
                    for eo in range(2):
                        for c in range(EMBED_DIM // 16):
                            sl = pl.ds(c * 16, 16)
                            plsc.addupdate(
                                bufs[b].at[eo, r, sl], pos_v[eo, r, sl]
                            )
                    return c2

                lax.fori_loop(0, half, add_body, None)
                out_start(b, g)

            @pl.when(i < n_groups - 1)
            def _():
                for b in range(NBUF):
                    g = go + b
                    out_wait(b, g)
                    gather_start(b, g + NBUF)

            return carry

        lax.fori_loop(0, n_groups, group_body, None)
        for b in range(NBUF):
            out_wait(b, bat_per_w - NBUF + b)

    return k


@jax.jit
def kernel(input_ids, token_table, position_table):
    batch, seq = input_ids.shape
    vocab = token_table.shape[0]
    half = seq // 2
    ids2 = input_ids.astype(jnp.int32) * 2
    ids_eo = ids2.reshape(batch, half, 2).transpose(0, 2, 1)
    tok2 = jnp.pad(token_table, ((0, 0), (0, EMBED_DIM))).reshape(
        2 * vocab, EMBED_DIM
    )
    posf = position_table[:seq].astype(jnp.float32)
    pos = jnp.stack([posf[0::2], posf[1::2]], axis=0)
    out = _make_sc_kernel(batch, seq)(ids_eo, tok2, pos)
    return out.reshape(batch, seq, EMBED_DIM)


# speedup vs baseline: 1.1715x; 1.1702x over previous
import functools

import jax
import jax.numpy as jnp
from jax import lax
from jax.experimental.layout import Format, Layout, with_layout_constraint
from jax.experimental import pallas as pl
from jax.experimental.pallas import tpu as pltpu
from jax.experimental.pallas import tpu_sc as plsc

EMBED_DIM = 64
NUM_WORKERS = 32
NBUF = 4


@functools.lru_cache(maxsize=None)
def _make_sc_kernel(batch: int, seq: int):
    bat_per_w = batch // NUM_WORKERS
    n_groups = bat_per_w // NBUF
    half = seq // 2
    mesh = plsc.VectorSubcoreMesh(core_axis_name="c", subcore_axis_name="s")

    @functools.partial(
        pl.kernel,
        mesh=mesh,
        compiler_params=pltpu.CompilerParams(use_tc_tiling_on_sc=False),
        out_type=jax.ShapeDtypeStruct((batch, half, 2 * EMBED_DIM), jnp.float32),
        scratch_types=[
            pltpu.VMEM((bat_per_w, 2, half), jnp.int32),
            pltpu.VMEM((2, half, EMBED_DIM), jnp.float32),
        ]
        + [pltpu.VMEM((2, half, EMBED_DIM), jnp.float32) for _ in range(NBUF)]
        + [pltpu.SemaphoreType.DMA for _ in range(2 * NBUF)],
    )
    def k(ids_hbm, tok_hbm, pos_hbm, out_hbm, idx_v, pos_v, *rest):
        bufs = rest[:NBUF]
        sg = rest[NBUF : 2 * NBUF]
        so = rest[2 * NBUF : 3 * NBUF]
        wid = lax.axis_index("s") * 2 + lax.axis_index("c")
        bat0 = wid * bat_per_w
        pltpu.sync_copy(pos_hbm, pos_v)
        pltpu.sync_copy(ids_hbm.at[pl.ds(bat0, bat_per_w)], idx_v)

        def gather_start(b, g):
            pltpu.async_copy(tok_hbm.at[idx_v.at[g, 0]], bufs[b].at[0], sg[b])
            pltpu.async_copy(tok_hbm.at[idx_v.at[g, 1]], bufs[b].at[1], sg[b])

        def gather_wait(b, g):
            pltpu.make_async_copy(
                tok_hbm.at[idx_v.at[g, 0]], bufs[b].at[0], sg[b]
            ).wait()
            pltpu.make_async_copy(
                tok_hbm.at[idx_v.at[g, 1]], bufs[b].at[1], sg[b]
            ).wait()

        def out_start(b, g):
            pltpu.async_copy(
                bufs[b].at[0], out_hbm.at[bat0 + g, :, pl.ds(0, EMBED_DIM)], so[b]
            )
            pltpu.async_copy(
                bufs[b].at[1],
                out_hbm.at[bat0 + g, :, pl.ds(EMBED_DIM, EMBED_DIM)],
                so[b],
            )

        def out_wait(b, g):
            pltpu.make_async_copy(
                bufs[b].at[0], out_hbm.at[bat0 + g, :, pl.ds(0, EMBED_DIM)], so[b]
            ).wait()
            pltpu.make_async_copy(
                bufs[b].at[1],
                out_hbm.at[bat0 + g, :, pl.ds(EMBED_DIM, EMBED_DIM)],
                so[b],
            ).wait()

        for b in range(NBUF):
            gather_start(b, b)

        def group_body(i, carry):
            go = i * NBUF
            for b in range(NBUF):
                g = go + b
                gather_wait(b, g)

                def add_body(r, c2, b=b):
                    for eo in range(2):
                        for c in range(EMBED_DIM // 16):
                            sl = pl.ds(c * 16, 16)
                            plsc.addupdate(
                                bufs[b].at[eo, r, sl], pos_v[eo, r, sl]
                            )
                    return c2

                lax.fori_loop(0, half, add_body, None)
                out_start(b, g)

            @pl.when(i < n_groups - 1)
            def _():
                for b in range(NBUF):
                    g = go + b
                    out_wait(b, g)
                    gather_start(b, g + NBUF)

            return carry

        lax.fori_loop(0, n_groups, group_body, None)
        for b in range(NBUF):
            out_wait(b, bat_per_w - NBUF + b)

    return k


@jax.jit
def kernel(input_ids, token_table, position_table):
    batch, seq = input_ids.shape
    vocab = token_table.shape[0]
    half = seq // 2
    ids2 = input_ids.astype(jnp.int32)
    ids_eo = ids2.reshape(batch, half, 2).transpose(0, 2, 1)
    tokc = with_layout_constraint(token_table, Layout((0, 1), ((8,),)))
    posf = position_table[:seq].astype(jnp.float32)
    pos = jnp.stack([posf[0::2], posf[1::2]], axis=0)
    out = _make_sc_kernel(batch, seq)(ids_eo, tokc, pos)
    outc = with_layout_constraint(out, Layout((0, 1, 2), ((8,),)))
    return outc.reshape(batch, seq, EMBED_DIM)
